# Initial kernel scaffold; baseline (speedup 1.0000x reference)
import jax
import jax.numpy as jnp
from jax.experimental import pallas as pl


def kernel(x, W_lin, b_lin, V0, g0, b0, V1, g1, b1, src0, dst0, src1, dst1):
    raise NotImplementedError("write your pallas kernel here")


# speedup vs baseline: 1.3869x; 1.3869x over previous
import functools

import jax
import jax.numpy as jnp
from jax import lax
from jax.experimental import pallas as pl
from jax.experimental.pallas import tpu as pltpu
from jax.experimental.pallas import tpu_sc as plsc

B = 256
K = 768
C2 = 128
N2 = 32
N1 = 512
N0 = 65536
E0 = 1024
E1 = 131072
NC = 2
NS = 16
NW = NC * NS
BPW = B // NW
CHUNK = 8192
NCHUNK = E1 // CHUNK


def _tc_body(x_ref, wl_ref, bl_ref, v0_ref, g0_ref, b0_ref, v1_ref, g1_ref,
             b1_ref, s0_ref, d0_ref, s1_ref, d1_ref,
             u_ref, c_ref, p0_ref, p1_ref):
    V0 = v0_ref[...]
    n0 = jnp.sqrt(jnp.sum(V0 * V0, axis=1, keepdims=True)) + 1e-12
    W0 = (g0_ref[...] / n0) * V0
    V1 = v1_ref[...]
    n1 = jnp.sqrt(jnp.sum(V1 * V1)) + 1e-12
    W1 = (g1_ref[0, 0] / n1) * V1
    w01 = jnp.dot(W1, W0, preferred_element_type=jnp.float32)

    j128 = lax.broadcasted_iota(jnp.int32, (C2 * N2, C2), 0)
    c128 = lax.broadcasted_iota(jnp.int32, (C2 * N2, C2), 1)
    onehot = jnp.where((j128 >> 5) == c128, 1.0, 0.0)
    w01rep = lax.dot_general(onehot, w01, (((1,), (1,)), ((), ())),
                             preferred_element_type=jnp.float32)
    jj = lax.broadcasted_iota(jnp.int32, (C2 * N2, N2), 0)
    nn = lax.broadcasted_iota(jnp.int32, (C2 * N2, N2), 1)
    M = jnp.where((jj & 31) == nn, w01rep, 0.0)

    Wu = lax.dot_general(wl_ref[...], M, (((0,), (0,)), ((), ())),
                         preferred_element_type=jnp.float32)
    ub = jnp.dot(bl_ref[...], M, preferred_element_type=jnp.float32)
    u_ref[...] = jnp.dot(x_ref[...], Wu,
                         preferred_element_type=jnp.float32) + ub

    C0 = jnp.sum(W1 * b0_ref[...])
    row = lax.broadcasted_iota(jnp.int32, (2, 128), 0)
    c_ref[...] = jnp.where(row == 0, C0, b1_ref[0, 0])

    p0_ref[...] = jnp.bitwise_or(lax.shift_left(d0_ref[...], 5), s0_ref[...])
    p1_ref[...] = jnp.bitwise_or(lax.shift_left(d1_ref[...], 9), s1_ref[...])


@jax.jit
def _tc_prep(x, wl, bl, v0, g0, b0, v1, g1, b1, s0, d0, s1, d1):
    return pl.pallas_call(
        _tc_body,
        out_shape=[
            jax.ShapeDtypeStruct((B, N2), jnp.float32),
            jax.ShapeDtypeStruct((2, 128), jnp.float32),
            jax.ShapeDtypeStruct((E0 // 128, 128), jnp.int32),
            jax.ShapeDtypeStruct((E1 // 128, 128), jnp.int32),
        ],
    )(x, wl, bl, v0, g0, b0, v1, g1, b1, s0, d0, s1, d1)


def _sc_body(u_hbm, c_hbm, p0_hbm, p1_hbm, out_hbm,
             u_loc, y8, p0_v, p1a, p1b, acc, cv, sem_in):
    wid = lax.axis_index("s") * NC + lax.axis_index("c")
    b_base = wid * BPW

    pltpu.sync_copy(u_hbm.at[pl.ds(wid * (BPW * N2), BPW * N2)], u_loc)
    pltpu.sync_copy(c_hbm, cv)
    pltpu.sync_copy(p0_hbm, p0_v)
    c0 = cv[pl.ds(0, 16)]
    b1v = cv[pl.ds(16, 16)]

    @pl.loop(0, BPW * N1 // 16)
    def _init_y(i):
        y8[pl.ds(i * 16, 16)] = c0

    @pl.loop(0, E0 // 16)
    def _fgl0(i):
        pk = p0_v[pl.ds(i * 16, 16)]
        s = jnp.bitwise_and(pk, 31)
        d = lax.shift_right_logical(pk, 5)
        for b in range(BPW):
            vals = plsc.load_gather(u_loc, [s + b * N2])
            plsc.addupdate_scatter(y8, [d + b * N1], vals)

    for b in range(BPW):
        @pl.loop(0, N0 // 16)
        def _init_acc(i):
            acc[pl.ds(i * 16, 16)] = b1v

        bufs = [p1a, p1b]
        descs = [None, None]
        descs[0] = pltpu.async_copy(p1_hbm.at[pl.ds(0, CHUNK)], bufs[0],
                                    sem_in)
        for c in range(NCHUNK):
            cur = c & 1
            descs[cur].wait()
            if c + 1 < NCHUNK:
                descs[1 - cur] = pltpu.async_copy(
                    p1_hbm.at[pl.ds((c + 1) * CHUNK, CHUNK)],
                    bufs[1 - cur], sem_in)
            buf = bufs[cur]

            @pl.loop(0, CHUNK // 16, unroll=8)
            def _edges(t):
                pk = buf[pl.ds(t * 16, 16)]
                s = jnp.bitwise_and(pk, 511)
                d = lax.shift_right_logical(pk, 9)
                vals = plsc.load_gather(y8, [s + b * N1])
                plsc.addupdate_scatter(acc, [d], vals)

        pltpu.sync_copy(acc, out_hbm.at[b_base + b])


def _make_sc():
    mesh = plsc.VectorSubcoreMesh(core_axis_name="c", subcore_axis_name="s",
                                  num_cores=NC, num_subcores=NS)
    return pl.kernel(
        _sc_body,
        out_type=jax.ShapeDtypeStruct((B, N0), jnp.float32),
        mesh=mesh,
        compiler_params=pltpu.CompilerParams(needs_layout_passes=False),
        scratch_types=[
            pltpu.VMEM((B * N2 // NW,), jnp.float32),
            pltpu.VMEM((BPW * N1,), jnp.float32),
            pltpu.VMEM((E0,), jnp.int32),
            pltpu.VMEM((CHUNK,), jnp.int32),
            pltpu.VMEM((CHUNK,), jnp.int32),
            pltpu.VMEM((N0,), jnp.float32),
            pltpu.VMEM((32,), jnp.float32),
            pltpu.SemaphoreType.DMA,
        ],
    )


def kernel(x, W_lin, b_lin, V0, g0, b0, V1, g1, b1, src0, dst0, src1, dst1):
    u, consts, p0, p1 = _tc_prep(
        x, W_lin, b_lin.reshape(1, -1), V0, g0.reshape(-1, 1),
        b0.reshape(1, -1), V1, g1.reshape(1, 1), b1.reshape(1, 1),
        src0.reshape(E0 // 128, 128), dst0.reshape(E0 // 128, 128),
        src1.reshape(E1 // 128, 128), dst1.reshape(E1 // 128, 128))
    cflat = jnp.concatenate([consts[0, :16], consts[1, :16]])
    return _make_sc()(u.reshape(-1), cflat, p0.reshape(-1), p1.reshape(-1))


# speedup vs baseline: 4.2743x; 3.0820x over previous
import functools

import jax
import jax.numpy as jnp
from jax import lax
from jax.experimental import pallas as pl
from jax.experimental.pallas import tpu as pltpu
from jax.experimental.pallas import tpu_sc as plsc

B = 256
K = 768
C2 = 128
N2 = 32
N1 = 512
N0 = 65536
E0 = 1024
E1 = 131072
NC = 2
NS = 16
NW = NC * NS
BPW = B // NW
CHUNK = 8192
NCHUNK = E1 // CHUNK


def _tc_body(x_ref, wl_ref, bl_ref, v0_ref, g0_ref, b0_ref, v1_ref, g1_ref,
             b1_ref, s0_ref, d0_ref, s1_ref, d1_ref,
             u_ref, c_ref, p0_ref, p1_ref):
    V0 = v0_ref[...]
    n0 = jnp.sqrt(jnp.sum(V0 * V0, axis=1, keepdims=True)) + 1e-12
    W0 = (g0_ref[...] / n0) * V0
    V1 = v1_ref[...]
    n1 = jnp.sqrt(jnp.sum(V1 * V1)) + 1e-12
    W1 = (g1_ref[0, 0] / n1) * V1
    w01 = jnp.dot(W1, W0, preferred_element_type=jnp.float32)

    j128 = lax.broadcasted_iota(jnp.int32, (C2 * N2, C2), 0)
    c128 = lax.broadcasted_iota(jnp.int32, (C2 * N2, C2), 1)
    onehot = jnp.where((j128 >> 5) == c128, 1.0, 0.0)
    w01rep = lax.dot_general(onehot, w01, (((1,), (1,)), ((), ())),
                             preferred_element_type=jnp.float32)
    jj = lax.broadcasted_iota(jnp.int32, (C2 * N2, N2), 0)
    nn = lax.broadcasted_iota(jnp.int32, (C2 * N2, N2), 1)
    M = jnp.where((jj & 31) == nn, w01rep, 0.0)

    Wu = lax.dot_general(wl_ref[...], M, (((0,), (0,)), ((), ())),
                         preferred_element_type=jnp.float32)
    ub = jnp.dot(bl_ref[...], M, preferred_element_type=jnp.float32)
    u_ref[...] = jnp.dot(x_ref[...], Wu,
                         preferred_element_type=jnp.float32) + ub

    C0 = jnp.sum(W1 * b0_ref[...])
    row = lax.broadcasted_iota(jnp.int32, (2, 128), 0)
    c_ref[...] = jnp.where(row == 0, C0, b1_ref[0, 0])

    p0_ref[...] = jnp.bitwise_or(lax.shift_left(d0_ref[...], 5), s0_ref[...])
    p1_ref[...] = jnp.bitwise_or(lax.shift_left(d1_ref[...], 9), s1_ref[...])


@jax.jit
def _tc_prep(x, wl, bl, v0, g0, b0, v1, g1, b1, s0, d0, s1, d1):
    return pl.pallas_call(
        _tc_body,
        out_shape=[
            jax.ShapeDtypeStruct((B, N2), jnp.float32),
            jax.ShapeDtypeStruct((2, 128), jnp.float32),
            jax.ShapeDtypeStruct((E0 // 128, 128), jnp.int32),
            jax.ShapeDtypeStruct((E1 // 128, 128), jnp.int32),
        ],
    )(x, wl, bl, v0, g0, b0, v1, g1, b1, s0, d0, s1, d1)


def _sc_body(u_hbm, c_hbm, p0_hbm, p1_hbm, out_hbm,
             u_loc, y8, p0_v, p1a, p1b, acc, cv, sem_in):
    wid = lax.axis_index("s") * NC + lax.axis_index("c")
    b_base = wid * BPW

    pltpu.sync_copy(u_hbm.at[pl.ds(wid * (BPW * N2), BPW * N2)], u_loc)
    pltpu.sync_copy(c_hbm, cv)
    pltpu.sync_copy(p0_hbm, p0_v)
    c0 = cv[pl.ds(0, 16)]
    b1v = cv[pl.ds(16, 16)]

    @plsc.parallel_loop(0, BPW * N1 // 16, unroll=4)
    def _init_y(i):
        y8[pl.ds(i * 16, 16)] = c0

    @plsc.parallel_loop(0, E0 // 16, unroll=2)
    def _fgl0(i):
        pk = p0_v[pl.ds(i * 16, 16)]
        s = jnp.bitwise_and(pk, 31)
        d = lax.shift_right_logical(pk, 5)
        for b in range(BPW):
            vals = plsc.load_gather(u_loc, [s + b * N2])
            plsc.addupdate_scatter(y8, [d + b * N1], vals)

    for b in range(BPW):
        @plsc.parallel_loop(0, N0 // 16, unroll=4)
        def _init_acc(i):
            acc[pl.ds(i * 16, 16)] = b1v

        bufs = [p1a, p1b]
        descs = [None, None]
        descs[0] = pltpu.async_copy(p1_hbm.at[pl.ds(0, CHUNK)], bufs[0],
                                    sem_in)
        for c in range(NCHUNK):
            cur = c & 1
            descs[cur].wait()
            if c + 1 < NCHUNK:
                descs[1 - cur] = pltpu.async_copy(
                    p1_hbm.at[pl.ds((c + 1) * CHUNK, CHUNK)],
                    bufs[1 - cur], sem_in)
            buf = bufs[cur]

            @plsc.parallel_loop(0, CHUNK // 16, unroll=8)
            def _edges(t):
                pk = buf[pl.ds(t * 16, 16)]
                s = jnp.bitwise_and(pk, 511)
                d = lax.shift_right_logical(pk, 9)
                vals = plsc.load_gather(y8, [s + b * N1])
                plsc.addupdate_scatter(acc, [d], vals)

        pltpu.sync_copy(acc, out_hbm.at[b_base + b])


def _make_sc():
    mesh = plsc.VectorSubcoreMesh(core_axis_name="c", subcore_axis_name="s",
                                  num_cores=NC, num_subcores=NS)
    return pl.kernel(
        _sc_body,
        out_type=jax.ShapeDtypeStruct((B, N0), jnp.float32),
        mesh=mesh,
        compiler_params=pltpu.CompilerParams(needs_layout_passes=False),
        scratch_types=[
            pltpu.VMEM((B * N2 // NW,), jnp.float32),
            pltpu.VMEM((BPW * N1,), jnp.float32),
            pltpu.VMEM((E0,), jnp.int32),
            pltpu.VMEM((CHUNK,), jnp.int32),
            pltpu.VMEM((CHUNK,), jnp.int32),
            pltpu.VMEM((N0,), jnp.float32),
            pltpu.VMEM((32,), jnp.float32),
            pltpu.SemaphoreType.DMA,
        ],
    )


def kernel(x, W_lin, b_lin, V0, g0, b0, V1, g1, b1, src0, dst0, src1, dst1):
    u, consts, p0, p1 = _tc_prep(
        x, W_lin, b_lin.reshape(1, -1), V0, g0.reshape(-1, 1),
        b0.reshape(1, -1), V1, g1.reshape(1, 1), b1.reshape(1, 1),
        src0.reshape(E0 // 128, 128), dst0.reshape(E0 // 128, 128),
        src1.reshape(E1 // 128, 128), dst1.reshape(E1 // 128, 128))
    cflat = jnp.concatenate([consts[0, :16], consts[1, :16]])
    return _make_sc()(u.reshape(-1), cflat, p0.reshape(-1), p1.reshape(-1))


# speedup vs baseline: 4.2787x; 1.0010x over previous
import functools

import jax
import jax.numpy as jnp
from jax import lax
from jax.experimental import pallas as pl
from jax.experimental.pallas import tpu as pltpu
from jax.experimental.pallas import tpu_sc as plsc

B = 256
K = 768
C2 = 128
N2 = 32
N1 = 512
N0 = 65536
E0 = 1024
E1 = 131072
NC = 2
NS = 16
NW = NC * NS
BPW = B // NW
CHUNK = 16384
NCHUNK = E1 // CHUNK


def _tc_body(x_ref, wl_ref, bl_ref, v0_ref, g0_ref, b0_ref, v1_ref, g1_ref,
             b1_ref, s0_ref, d0_ref, s1_ref, d1_ref,
             u_ref, c_ref, p0_ref, p1_ref):
    V0 = v0_ref[...]
    n0 = jnp.sqrt(jnp.sum(V0 * V0, axis=1, keepdims=True)) + 1e-12
    W0 = (g0_ref[...] / n0) * V0
    V1 = v1_ref[...]
    n1 = jnp.sqrt(jnp.sum(V1 * V1)) + 1e-12
    W1 = (g1_ref[0, 0] / n1) * V1
    w01 = jnp.dot(W1, W0, preferred_element_type=jnp.float32)

    j128 = lax.broadcasted_iota(jnp.int32, (C2 * N2, C2), 0)
    c128 = lax.broadcasted_iota(jnp.int32, (C2 * N2, C2), 1)
    onehot = jnp.where((j128 >> 5) == c128, 1.0, 0.0)
    w01rep = lax.dot_general(onehot, w01, (((1,), (1,)), ((), ())),
                             preferred_element_type=jnp.float32)
    jj = lax.broadcasted_iota(jnp.int32, (C2 * N2, N2), 0)
    nn = lax.broadcasted_iota(jnp.int32, (C2 * N2, N2), 1)
    M = jnp.where((jj & 31) == nn, w01rep, 0.0)

    Wu = lax.dot_general(wl_ref[...], M, (((0,), (0,)), ((), ())),
                         preferred_element_type=jnp.float32)
    ub = jnp.dot(bl_ref[...], M, preferred_element_type=jnp.float32)
    u_ref[...] = jnp.dot(x_ref[...], Wu,
                         preferred_element_type=jnp.float32) + ub

    C0 = jnp.sum(W1 * b0_ref[...])
    row = lax.broadcasted_iota(jnp.int32, (2, 128), 0)
    c_ref[...] = jnp.where(row == 0, C0, b1_ref[0, 0])

    p0_ref[...] = jnp.bitwise_or(lax.shift_left(d0_ref[...], 5), s0_ref[...])
    p1_ref[...] = jnp.bitwise_or(lax.shift_left(d1_ref[...], 9), s1_ref[...])


@jax.jit
def _tc_prep(x, wl, bl, v0, g0, b0, v1, g1, b1, s0, d0, s1, d1):
    return pl.pallas_call(
        _tc_body,
        out_shape=[
            jax.ShapeDtypeStruct((B, N2), jnp.float32),
            jax.ShapeDtypeStruct((2, 128), jnp.float32),
            jax.ShapeDtypeStruct((E0 // 128, 128), jnp.int32),
            jax.ShapeDtypeStruct((E1 // 128, 128), jnp.int32),
        ],
    )(x, wl, bl, v0, g0, b0, v1, g1, b1, s0, d0, s1, d1)


def _sc_body(u_hbm, c_hbm, p0_hbm, p1_hbm, out_hbm,
             u_loc, y8, p0_v, p1a, p1b, acc, cv, sem_in):
    wid = lax.axis_index("s") * NC + lax.axis_index("c")
    b_base = wid * BPW

    pltpu.sync_copy(u_hbm.at[pl.ds(wid * (BPW * N2), BPW * N2)], u_loc)
    pltpu.sync_copy(c_hbm, cv)
    pltpu.sync_copy(p0_hbm, p0_v)
    c0 = cv[pl.ds(0, 16)]
    b1v = cv[pl.ds(16, 16)]

    @plsc.parallel_loop(0, BPW * N1 // 16, unroll=4)
    def _init_y(i):
        y8[pl.ds(i * 16, 16)] = c0

    @plsc.parallel_loop(0, E0 // 16, unroll=2)
    def _fgl0(i):
        pk = p0_v[pl.ds(i * 16, 16)]
        s = jnp.bitwise_and(pk, 31)
        d = lax.shift_right_logical(pk, 5)
        for b in range(BPW):
            vals = plsc.load_gather(u_loc, [s + b * N2])
            plsc.addupdate_scatter(y8, [d + b * N1], vals)

    for b in range(BPW):
        @plsc.parallel_loop(0, N0 // 16, unroll=4)
        def _init_acc(i):
            acc[pl.ds(i * 16, 16)] = b1v

        bufs = [p1a, p1b]
        descs = [None, None]
        descs[0] = pltpu.async_copy(p1_hbm.at[pl.ds(0, CHUNK)], bufs[0],
                                    sem_in)
        for c in range(NCHUNK):
            cur = c & 1
            descs[cur].wait()
            if c + 1 < NCHUNK:
                descs[1 - cur] = pltpu.async_copy(
                    p1_hbm.at[pl.ds((c + 1) * CHUNK, CHUNK)],
                    bufs[1 - cur], sem_in)
            buf = bufs[cur]

            @plsc.parallel_loop(0, CHUNK // 16, unroll=8)
            def _edges(t):
                pk = buf[pl.ds(t * 16, 16)]
                s = jnp.bitwise_and(pk, 511)
                d = lax.shift_right_logical(pk, 9)
                vals = plsc.load_gather(y8, [s + b * N1])
                plsc.addupdate_scatter(acc, [d], vals)

        pltpu.sync_copy(acc, out_hbm.at[b_base + b])


def _make_sc():
    mesh = plsc.VectorSubcoreMesh(core_axis_name="c", subcore_axis_name="s",
                                  num_cores=NC, num_subcores=NS)
    return pl.kernel(
        _sc_body,
        out_type=jax.ShapeDtypeStruct((B, N0), jnp.float32),
        mesh=mesh,
        compiler_params=pltpu.CompilerParams(needs_layout_passes=False),
        scratch_types=[
            pltpu.VMEM((B * N2 // NW,), jnp.float32),
            pltpu.VMEM((BPW * N1,), jnp.float32),
            pltpu.VMEM((E0,), jnp.int32),
            pltpu.VMEM((CHUNK,), jnp.int32),
            pltpu.VMEM((CHUNK,), jnp.int32),
            pltpu.VMEM((N0,), jnp.float32),
            pltpu.VMEM((32,), jnp.float32),
            pltpu.SemaphoreType.DMA,
        ],
    )


def kernel(x, W_lin, b_lin, V0, g0, b0, V1, g1, b1, src0, dst0, src1, dst1):
    u, consts, p0, p1 = _tc_prep(
        x, W_lin, b_lin.reshape(1, -1), V0, g0.reshape(-1, 1),
        b0.reshape(1, -1), V1, g1.reshape(1, 1), b1.reshape(1, 1),
        src0.reshape(E0 // 128, 128), dst0.reshape(E0 // 128, 128),
        src1.reshape(E1 // 128, 128), dst1.reshape(E1 // 128, 128))
    cflat = jnp.concatenate([consts[0, :16], consts[1, :16]])
    return _make_sc()(u.reshape(-1), cflat, p0.reshape(-1), p1.reshape(-1))
